# baseline (speedup 1.0000x reference)
import functools

import jax
import jax.numpy as jnp
from jax import lax
from jax.experimental import pallas as pl
from jax.experimental.pallas import tpu as pltpu
from jax.experimental.pallas import tpu_sc as plsc

N = 10000
E = 320000
D_IN = 128
HIDDEN = 64
EMB = 2
G = 16

NC = 2
NS = 16
NW = NC * NS
NPAD = 10240
RPT = NPAD // NS
CH = 128
CHUNKS = 80
EPW = CH * CHUNKS
EPAD = EPW * NW
RING = 4
HALF = RING // 2


def _deg_body(dst_hbm, out_hbm, idx_v, ones_v, zb_v, acc, sem):
    cid = lax.axis_index("c")
    sid = lax.axis_index("s")
    wid = sid * NC + cid

    @pl.loop(0, RPT // 16)
    def _zero(i):
        zb_v[pl.ds(i * 16, 16)] = jnp.zeros((16,), jnp.float32)

    for i in range(CH // 16):
        ones_v[pl.ds(i * 16, 16)] = jnp.ones((16,), jnp.float32)

    pltpu.sync_copy(zb_v, acc.at[pl.ds(sid * RPT, RPT)])
    plsc.subcore_barrier()

    pltpu.async_copy(dst_hbm.at[wid], idx_v, sem).wait()

    @pl.loop(0, CHUNKS)
    def _edges(j):
        pltpu.sync_copy(ones_v, acc.at[idx_v.at[j]], add=True)

    plsc.subcore_barrier()
    pltpu.sync_copy(acc.at[pl.ds(sid * RPT, RPT)],
                    out_hbm.at[cid, 0, pl.ds(sid * RPT, RPT)])


@functools.cache
def _deg_call():
    return pl.kernel(
        _deg_body,
        out_type=jax.ShapeDtypeStruct((NC, 1, NPAD), jnp.float32),
        mesh=plsc.VectorSubcoreMesh(core_axis_name="c", subcore_axis_name="s",
                                    num_cores=NC, num_subcores=NS),
        scratch_types=[
            pltpu.VMEM((CHUNKS, CH), jnp.int32),
            pltpu.VMEM((CH,), jnp.float32),
            pltpu.VMEM((RPT,), jnp.float32),
            pltpu.VMEM_SHARED((NPAD,), jnp.float32),
            pltpu.SemaphoreType.DMA,
        ],
        compiler_params=pltpu.CompilerParams(use_tc_tiling_on_sc=False),
    )


def _agg_body(d, g_hbm, src_hbm, dst_hbm, out_hbm,
              idx_s, idx_d, rows, zb_v, acc, gsem, isem):
    cid = lax.axis_index("c")
    sid = lax.axis_index("s")
    wid = sid * NC + cid

    @pl.loop(0, CH)
    def _zero(i):
        for k in range(d // 16):
            zb_v[i, pl.ds(k * 16, 16)] = jnp.zeros((16,), jnp.float32)

    @pl.loop(0, RPT // CH)
    def _zacc(k):
        pltpu.sync_copy(zb_v, acc.at[pl.ds(sid * RPT + k * CH, CH)])

    plsc.subcore_barrier()

    pltpu.async_copy(src_hbm.at[wid], idx_s, isem)
    pltpu.async_copy(dst_hbm.at[wid], idx_d, isem)
    pltpu.make_async_copy(src_hbm.at[wid], idx_s, isem).wait()
    pltpu.make_async_copy(dst_hbm.at[wid], idx_d, isem).wait()

    for b in range(RING):
        pltpu.async_copy(g_hbm.at[idx_s.at[b]], rows.at[b], gsem.at[b])

    @pl.loop(0, CHUNKS, step=RING)
    def _edges(j0):
        for b in range(RING):
            j = j0 + b
            pltpu.make_async_copy(g_hbm.at[idx_s.at[b]], rows.at[b],
                                  gsem.at[b]).wait()
            pltpu.sync_copy(rows.at[b], acc.at[idx_d.at[j]], add=True)

            @pl.when(j + RING < CHUNKS)
            def _ng():
                pltpu.async_copy(g_hbm.at[idx_s.at[j + RING]], rows.at[b],
                                 gsem.at[b])

    plsc.subcore_barrier()
    pltpu.sync_copy(acc.at[pl.ds(sid * RPT, RPT)],
                    out_hbm.at[cid, pl.ds(sid * RPT, RPT)])


@functools.cache
def _make_agg(d):
    return pl.kernel(
        functools.partial(_agg_body, d),
        out_type=jax.ShapeDtypeStruct((NC, NPAD, d), jnp.float32),
        mesh=plsc.VectorSubcoreMesh(core_axis_name="c", subcore_axis_name="s",
                                    num_cores=NC, num_subcores=NS),
        scratch_types=[
            pltpu.VMEM((CHUNKS, CH), jnp.int32),
            pltpu.VMEM((CHUNKS, CH), jnp.int32),
            pltpu.VMEM((RING, CH, d), jnp.float32),
            pltpu.VMEM((CH, d), jnp.float32),
            pltpu.VMEM_SHARED((NPAD, d), jnp.float32),
            pltpu.SemaphoreType.DMA((RING,)),
            pltpu.SemaphoreType.DMA,
        ],
        compiler_params=pltpu.CompilerParams(use_tc_tiling_on_sc=False),
    )


def _agg2e_body(ga_hbm, gb_hbm, src_hbm, dst_hbm, oa_hbm, ob_hbm,
                idx_s, idx_d, va, vb, zb_v, acc_a, acc_b, gsa, gsb, isem):
    cid = lax.axis_index("c")
    sid = lax.axis_index("s")
    wid = sid * NC + cid

    @pl.loop(0, RPT // 16)
    def _zero(i):
        zb_v[pl.ds(i * 16, 16)] = jnp.zeros((16,), jnp.float32)

    pltpu.sync_copy(zb_v, acc_a.at[pl.ds(sid * RPT, RPT)])
    pltpu.sync_copy(zb_v, acc_b.at[pl.ds(sid * RPT, RPT)])
    plsc.subcore_barrier()

    pltpu.async_copy(src_hbm.at[wid], idx_s, isem)
    pltpu.async_copy(dst_hbm.at[wid], idx_d, isem)
    pltpu.make_async_copy(src_hbm.at[wid], idx_s, isem).wait()
    pltpu.make_async_copy(dst_hbm.at[wid], idx_d, isem).wait()

    for b in range(RING):
        pltpu.async_copy(ga_hbm.at[idx_s.at[b]], va.at[b], gsa.at[b])
        pltpu.async_copy(gb_hbm.at[idx_s.at[b]], vb.at[b], gsb.at[b])

    @pl.loop(0, CHUNKS, step=RING)
    def _edges(j0):
        for b in range(RING):
            j = j0 + b
            pltpu.make_async_copy(ga_hbm.at[idx_s.at[b]], va.at[b],
                                  gsa.at[b]).wait()
            pltpu.make_async_copy(gb_hbm.at[idx_s.at[b]], vb.at[b],
                                  gsb.at[b]).wait()
            pltpu.sync_copy(va.at[b], acc_a.at[idx_d.at[j]], add=True)
            pltpu.sync_copy(vb.at[b], acc_b.at[idx_d.at[j]], add=True)

            @pl.when(j + RING < CHUNKS)
            def _ng():
                pltpu.async_copy(ga_hbm.at[idx_s.at[j + RING]], va.at[b],
                                 gsa.at[b])
                pltpu.async_copy(gb_hbm.at[idx_s.at[j + RING]], vb.at[b],
                                 gsb.at[b])

    plsc.subcore_barrier()
    pltpu.sync_copy(acc_a.at[pl.ds(sid * RPT, RPT)],
                    oa_hbm.at[cid, 0, pl.ds(sid * RPT, RPT)])
    pltpu.sync_copy(acc_b.at[pl.ds(sid * RPT, RPT)],
                    ob_hbm.at[cid, 0, pl.ds(sid * RPT, RPT)])


@functools.cache
def _agg2e_call():
    return pl.kernel(
        _agg2e_body,
        out_type=[jax.ShapeDtypeStruct((NC, 1, NPAD), jnp.float32),
                  jax.ShapeDtypeStruct((NC, 1, NPAD), jnp.float32)],
        mesh=plsc.VectorSubcoreMesh(core_axis_name="c", subcore_axis_name="s",
                                    num_cores=NC, num_subcores=NS),
        scratch_types=[
            pltpu.VMEM((CHUNKS, CH), jnp.int32),
            pltpu.VMEM((CHUNKS, CH), jnp.int32),
            pltpu.VMEM((RING, CH), jnp.float32),
            pltpu.VMEM((RING, CH), jnp.float32),
            pltpu.VMEM((RPT,), jnp.float32),
            pltpu.VMEM_SHARED((NPAD,), jnp.float32),
            pltpu.VMEM_SHARED((NPAD,), jnp.float32),
            pltpu.SemaphoreType.DMA((RING,)),
            pltpu.SemaphoreType.DMA((RING,)),
            pltpu.SemaphoreType.DMA,
        ],
        compiler_params=pltpu.CompilerParams(use_tc_tiling_on_sc=False),
    )


BLK = 1024


def _b1_body(x_ref, w1_ref, h_ref):
    h_ref[...] = jnp.dot(x_ref[...], w1_ref[...],
                         preferred_element_type=jnp.float32)


def _b1_call(x_p, W1):
    return pl.pallas_call(
        _b1_body,
        grid=(NPAD // BLK,),
        in_specs=[
            pl.BlockSpec((BLK, D_IN), lambda i: (i, 0)),
            pl.BlockSpec((D_IN, HIDDEN), lambda i: (0, 0)),
        ],
        out_specs=pl.BlockSpec((BLK, HIDDEN), lambda i: (i, 0)),
        out_shape=jax.ShapeDtypeStruct((NPAD, HIDDEN), jnp.float32),
    )(x_p, W1)


def _b2_body(deg_ref, h_ref, g1_ref, dinv_ref):
    d = deg_ref[:, 0:1] + deg_ref[:, 1:2] + 1.0
    dinv = lax.rsqrt(d)
    g1_ref[...] = h_ref[...] * dinv
    dinv_ref[...] = dinv


def _b2_call(deg, h):
    return pl.pallas_call(
        _b2_body,
        grid=(NPAD // BLK,),
        in_specs=[
            pl.BlockSpec((BLK, NC), lambda i: (i, 0)),
            pl.BlockSpec((BLK, HIDDEN), lambda i: (i, 0)),
        ],
        out_specs=[
            pl.BlockSpec((BLK, HIDDEN), lambda i: (i, 0)),
            pl.BlockSpec((BLK, 1), lambda i: (i, 0)),
        ],
        out_shape=[
            jax.ShapeDtypeStruct((NPAD, HIDDEN), jnp.float32),
            jax.ShapeDtypeStruct((NPAD, 1), jnp.float32),
        ],
    )(deg, h)


def _d_body(p_ref, g1_ref, dinv_ref, w2_ref, b1_ref, g2t_ref):
    s = p_ref[0] + p_ref[1] + g1_ref[...]
    h1 = jnp.maximum(dinv_ref[...] * s + b1_ref[...], 0.0)
    u = h1 * dinv_ref[...]
    g2t_ref[...] = lax.dot_general(
        w2_ref[...], u, (((0,), (1,)), ((), ())),
        preferred_element_type=jnp.float32)


def _d_call(p, g1, dinv, W2, b1):
    return pl.pallas_call(
        _d_body,
        grid=(NPAD // BLK,),
        in_specs=[
            pl.BlockSpec((NC, BLK, HIDDEN), lambda i: (0, i, 0)),
            pl.BlockSpec((BLK, HIDDEN), lambda i: (i, 0)),
            pl.BlockSpec((BLK, 1), lambda i: (i, 0)),
            pl.BlockSpec((HIDDEN, EMB), lambda i: (0, 0)),
            pl.BlockSpec((1, HIDDEN), lambda i: (0, 0)),
        ],
        out_specs=pl.BlockSpec((EMB, BLK), lambda i: (0, i)),
        out_shape=jax.ShapeDtypeStruct((EMB, NPAD), jnp.float32),
    )(p, g1, dinv, W2, b1)


def _e_body(qa_ref, qb_ref, g2t_ref, dinvt_ref, batch_ref, b2_ref, out_ref,
            cnt_ref):
    i = pl.program_id(0)

    @pl.when(i == 0)
    def _init():
        out_ref[...] = jnp.zeros_like(out_ref)
        cnt_ref[...] = jnp.zeros_like(cnt_ref)

    dinvt = dinvt_ref[...]
    na = dinvt * (qa_ref[0] + qa_ref[1] + g2t_ref[0:1, :])
    nb = dinvt * (qb_ref[0] + qb_ref[1] + g2t_ref[1:2, :])
    ids = lax.broadcasted_iota(jnp.int32, (G, BLK), 0)
    m = (batch_ref[...] == ids).astype(jnp.float32)
    ra = jnp.sum(m * na, axis=1, keepdims=True)
    rb = jnp.sum(m * nb, axis=1, keepdims=True)
    out_ref[...] += jnp.concatenate([ra, rb], axis=1)
    cnt_ref[...] += jnp.sum(m, axis=1, keepdims=True)

    @pl.when(i == NPAD // BLK - 1)
    def _fin():
        cnt = cnt_ref[...]
        mean = out_ref[...] / jnp.maximum(cnt, 1.0)
        out_ref[...] = mean + b2_ref[...] * (cnt > 0.0)


def _e_call(qa, qb, g2t, dinv_t, batch_p, b2):
    return pl.pallas_call(
        _e_body,
        grid=(NPAD // BLK,),
        in_specs=[
            pl.BlockSpec((NC, 1, BLK), lambda i: (0, 0, i)),
            pl.BlockSpec((NC, 1, BLK), lambda i: (0, 0, i)),
            pl.BlockSpec((EMB, BLK), lambda i: (0, i)),
            pl.BlockSpec((1, BLK), lambda i: (0, i)),
            pl.BlockSpec((1, BLK), lambda i: (0, i)),
            pl.BlockSpec((1, EMB), lambda i: (0, 0)),
        ],
        out_specs=pl.BlockSpec((G, EMB), lambda i: (0, 0)),
        out_shape=jax.ShapeDtypeStruct((G, EMB), jnp.float32),
        scratch_shapes=[pltpu.VMEM((G, 1), jnp.float32)],
    )(qa, qb, g2t, dinv_t, batch_p, b2)


@jax.jit
def kernel(x, edge_index, batch, W1, b1, W2, b2):
    src = edge_index[0].astype(jnp.int32)
    dst = edge_index[1].astype(jnp.int32)
    pad = N + (jnp.arange(EPAD - E, dtype=jnp.int32) % (NPAD - N))
    src3 = jnp.concatenate([src, pad]).reshape(NW, CHUNKS, CH)
    dst3 = jnp.concatenate([dst, pad]).reshape(NW, CHUNKS, CH)
    x_p = jnp.pad(x.astype(jnp.float32), ((0, NPAD - N), (0, 0)))
    batch_p = jnp.pad(batch.astype(jnp.int32), (0, NPAD - N),
                      constant_values=G).reshape(1, NPAD)

    deg = _deg_call()(dst3)
    h = _b1_call(x_p, W1)
    deg_t = jnp.transpose(deg.reshape(NC, NPAD))
    g1, dinv = _b2_call(deg_t, h)
    p = _make_agg(HIDDEN)(g1, src3, dst3)
    g2t = _d_call(p, g1, dinv, W2, b1.reshape(1, HIDDEN))
    qa, qb = _agg2e_call()(g2t[0], g2t[1], src3, dst3)
    dinv_t = jnp.transpose(dinv)
    out = _e_call(qa, qb, g2t, dinv_t, batch_p, b2.reshape(1, EMB))
    return out


# speedup vs baseline: 1.0086x; 1.0086x over previous
import functools

import jax
import jax.numpy as jnp
from jax import lax
from jax.experimental import pallas as pl
from jax.experimental.pallas import tpu as pltpu
from jax.experimental.pallas import tpu_sc as plsc

N = 10000
E = 320000
D_IN = 128
HIDDEN = 64
EMB = 2
G = 16

NC = 2
NS = 16
NW = NC * NS
NPAD = 10240
RPT = NPAD // NS
CH = 128
CHUNKS = 80
EPW = CH * CHUNKS
EPAD = EPW * NW
RING = 4
HALF = RING // 2


def _deg_body(dst_hbm, out_hbm, idx_v, ones_v, zb_v, acc, sem):
    cid = lax.axis_index("c")
    sid = lax.axis_index("s")
    wid = sid * NC + cid

    @pl.loop(0, RPT // 16)
    def _zero(i):
        zb_v[pl.ds(i * 16, 16)] = jnp.zeros((16,), jnp.float32)

    for i in range(CH // 16):
        ones_v[pl.ds(i * 16, 16)] = jnp.ones((16,), jnp.float32)

    pltpu.sync_copy(zb_v, acc.at[pl.ds(sid * RPT, RPT)])
    plsc.subcore_barrier()

    pltpu.async_copy(dst_hbm.at[wid], idx_v, sem).wait()

    @pl.loop(0, CHUNKS)
    def _edges(j):
        pltpu.sync_copy(ones_v, acc.at[idx_v.at[j]], add=True)

    plsc.subcore_barrier()
    pltpu.sync_copy(acc.at[pl.ds(sid * RPT, RPT)],
                    out_hbm.at[cid, 0, pl.ds(sid * RPT, RPT)])


@functools.cache
def _deg_call():
    return pl.kernel(
        _deg_body,
        out_type=jax.ShapeDtypeStruct((NC, 1, NPAD), jnp.float32),
        mesh=plsc.VectorSubcoreMesh(core_axis_name="c", subcore_axis_name="s",
                                    num_cores=NC, num_subcores=NS),
        scratch_types=[
            pltpu.VMEM((CHUNKS, CH), jnp.int32),
            pltpu.VMEM((CH,), jnp.float32),
            pltpu.VMEM((RPT,), jnp.float32),
            pltpu.VMEM_SHARED((NPAD,), jnp.float32),
            pltpu.SemaphoreType.DMA,
        ],
        compiler_params=pltpu.CompilerParams(use_tc_tiling_on_sc=False),
    )


def _agg_body(d, g_hbm, src_hbm, dst_hbm, out_hbm,
              idx_s, idx_d, rows, zb_v, acc, gsem, isem):
    cid = lax.axis_index("c")
    sid = lax.axis_index("s")
    wid = sid * NC + cid

    @pl.loop(0, CH)
    def _zero(i):
        for k in range(d // 16):
            zb_v[i, pl.ds(k * 16, 16)] = jnp.zeros((16,), jnp.float32)

    @pl.loop(0, RPT // CH)
    def _zacc(k):
        pltpu.sync_copy(zb_v, acc.at[pl.ds(sid * RPT + k * CH, CH)])

    plsc.subcore_barrier()

    pltpu.async_copy(src_hbm.at[wid], idx_s, isem)
    pltpu.async_copy(dst_hbm.at[wid], idx_d, isem)
    pltpu.make_async_copy(src_hbm.at[wid], idx_s, isem).wait()
    pltpu.make_async_copy(dst_hbm.at[wid], idx_d, isem).wait()

    for b in range(RING):
        pltpu.async_copy(g_hbm.at[idx_s.at[b]], rows.at[b], gsem.at[b])

    @pl.loop(0, CHUNKS, step=RING)
    def _edges(j0):
        for b in range(RING):
            j = j0 + b
            pltpu.make_async_copy(g_hbm.at[idx_s.at[b]], rows.at[b],
                                  gsem.at[b]).wait()
            pltpu.sync_copy(rows.at[b], acc.at[idx_d.at[j]], add=True)

            @pl.when(j + RING < CHUNKS)
            def _ng():
                pltpu.async_copy(g_hbm.at[idx_s.at[j + RING]], rows.at[b],
                                 gsem.at[b])

    plsc.subcore_barrier()
    pltpu.sync_copy(acc.at[pl.ds(sid * RPT, RPT)],
                    out_hbm.at[cid, pl.ds(sid * RPT, RPT)])


@functools.cache
def _make_agg(d):
    return pl.kernel(
        functools.partial(_agg_body, d),
        out_type=jax.ShapeDtypeStruct((NC, NPAD, d), jnp.float32),
        mesh=plsc.VectorSubcoreMesh(core_axis_name="c", subcore_axis_name="s",
                                    num_cores=NC, num_subcores=NS),
        scratch_types=[
            pltpu.VMEM((CHUNKS, CH), jnp.int32),
            pltpu.VMEM((CHUNKS, CH), jnp.int32),
            pltpu.VMEM((RING, CH, d), jnp.float32),
            pltpu.VMEM((CH, d), jnp.float32),
            pltpu.VMEM_SHARED((NPAD, d), jnp.float32),
            pltpu.SemaphoreType.DMA((RING,)),
            pltpu.SemaphoreType.DMA,
        ],
        compiler_params=pltpu.CompilerParams(use_tc_tiling_on_sc=False),
    )


def _agg2e_body(ga_hbm, gb_hbm, src_hbm, dst_hbm, oa_hbm, ob_hbm,
                idx_s, idx_d, va, vb, zb_v, acc_a, acc_b, gsa, gsb, isem):
    cid = lax.axis_index("c")
    sid = lax.axis_index("s")
    wid = sid * NC + cid

    @pl.loop(0, RPT // 16)
    def _zero(i):
        zb_v[pl.ds(i * 16, 16)] = jnp.zeros((16,), jnp.float32)

    pltpu.sync_copy(zb_v, acc_a.at[pl.ds(sid * RPT, RPT)])
    pltpu.sync_copy(zb_v, acc_b.at[pl.ds(sid * RPT, RPT)])
    plsc.subcore_barrier()

    pltpu.async_copy(src_hbm.at[wid], idx_s, isem)
    pltpu.async_copy(dst_hbm.at[wid], idx_d, isem)
    pltpu.make_async_copy(src_hbm.at[wid], idx_s, isem).wait()
    pltpu.make_async_copy(dst_hbm.at[wid], idx_d, isem).wait()

    for b in range(RING):
        pltpu.async_copy(ga_hbm.at[idx_s.at[b]], va.at[b], gsa.at[b])
        pltpu.async_copy(gb_hbm.at[idx_s.at[b]], vb.at[b], gsb.at[b])

    @pl.loop(0, CHUNKS, step=RING)
    def _edges(j0):
        for b in range(RING):
            j = j0 + b
            pltpu.make_async_copy(ga_hbm.at[idx_s.at[b]], va.at[b],
                                  gsa.at[b]).wait()
            pltpu.make_async_copy(gb_hbm.at[idx_s.at[b]], vb.at[b],
                                  gsb.at[b]).wait()
            pltpu.sync_copy(va.at[b], acc_a.at[idx_d.at[j]], add=True)
            pltpu.sync_copy(vb.at[b], acc_b.at[idx_d.at[j]], add=True)

            @pl.when(j + RING < CHUNKS)
            def _ng():
                pltpu.async_copy(ga_hbm.at[idx_s.at[j + RING]], va.at[b],
                                 gsa.at[b])
                pltpu.async_copy(gb_hbm.at[idx_s.at[j + RING]], vb.at[b],
                                 gsb.at[b])

    plsc.subcore_barrier()
    pltpu.sync_copy(acc_a.at[pl.ds(sid * RPT, RPT)],
                    oa_hbm.at[cid, 0, pl.ds(sid * RPT, RPT)])
    pltpu.sync_copy(acc_b.at[pl.ds(sid * RPT, RPT)],
                    ob_hbm.at[cid, 0, pl.ds(sid * RPT, RPT)])


@functools.cache
def _agg2e_call():
    return pl.kernel(
        _agg2e_body,
        out_type=[jax.ShapeDtypeStruct((NC, 1, NPAD), jnp.float32),
                  jax.ShapeDtypeStruct((NC, 1, NPAD), jnp.float32)],
        mesh=plsc.VectorSubcoreMesh(core_axis_name="c", subcore_axis_name="s",
                                    num_cores=NC, num_subcores=NS),
        scratch_types=[
            pltpu.VMEM((CHUNKS, CH), jnp.int32),
            pltpu.VMEM((CHUNKS, CH), jnp.int32),
            pltpu.VMEM((RING, CH), jnp.float32),
            pltpu.VMEM((RING, CH), jnp.float32),
            pltpu.VMEM((RPT,), jnp.float32),
            pltpu.VMEM_SHARED((NPAD,), jnp.float32),
            pltpu.VMEM_SHARED((NPAD,), jnp.float32),
            pltpu.SemaphoreType.DMA((RING,)),
            pltpu.SemaphoreType.DMA((RING,)),
            pltpu.SemaphoreType.DMA,
        ],
        compiler_params=pltpu.CompilerParams(use_tc_tiling_on_sc=False),
    )


BLK = 1024


def _b_body(deg_ref, x_ref, w1_ref, g1_ref, dinv_ref):
    d = deg_ref[:, 0:1] + deg_ref[:, 1:2] + 1.0
    dinv = lax.rsqrt(d)
    h = jnp.dot(x_ref[...], w1_ref[...], preferred_element_type=jnp.float32)
    g1_ref[...] = h * dinv
    dinv_ref[...] = dinv


def _b_call(deg, x_p, W1):
    return pl.pallas_call(
        _b_body,
        grid=(NPAD // BLK,),
        in_specs=[
            pl.BlockSpec((BLK, NC), lambda i: (i, 0)),
            pl.BlockSpec((BLK, D_IN), lambda i: (i, 0)),
            pl.BlockSpec((D_IN, HIDDEN), lambda i: (0, 0)),
        ],
        out_specs=[
            pl.BlockSpec((BLK, HIDDEN), lambda i: (i, 0)),
            pl.BlockSpec((BLK, 1), lambda i: (i, 0)),
        ],
        out_shape=[
            jax.ShapeDtypeStruct((NPAD, HIDDEN), jnp.float32),
            jax.ShapeDtypeStruct((NPAD, 1), jnp.float32),
        ],
    )(deg, x_p, W1)


def _d_body(p_ref, g1_ref, dinv_ref, w2_ref, b1_ref, g2t_ref):
    s = p_ref[0] + p_ref[1] + g1_ref[...]
    h1 = jnp.maximum(dinv_ref[...] * s + b1_ref[...], 0.0)
    u = h1 * dinv_ref[...]
    g2t_ref[...] = lax.dot_general(
        w2_ref[...], u, (((0,), (1,)), ((), ())),
        preferred_element_type=jnp.float32)


def _d_call(p, g1, dinv, W2, b1):
    return pl.pallas_call(
        _d_body,
        grid=(NPAD // BLK,),
        in_specs=[
            pl.BlockSpec((NC, BLK, HIDDEN), lambda i: (0, i, 0)),
            pl.BlockSpec((BLK, HIDDEN), lambda i: (i, 0)),
            pl.BlockSpec((BLK, 1), lambda i: (i, 0)),
            pl.BlockSpec((HIDDEN, EMB), lambda i: (0, 0)),
            pl.BlockSpec((1, HIDDEN), lambda i: (0, 0)),
        ],
        out_specs=pl.BlockSpec((EMB, BLK), lambda i: (0, i)),
        out_shape=jax.ShapeDtypeStruct((EMB, NPAD), jnp.float32),
    )(p, g1, dinv, W2, b1)


def _e_body(qa_ref, qb_ref, g2t_ref, dinvt_ref, batch_ref, b2_ref, out_ref,
            cnt_ref):
    i = pl.program_id(0)

    @pl.when(i == 0)
    def _init():
        out_ref[...] = jnp.zeros_like(out_ref)
        cnt_ref[...] = jnp.zeros_like(cnt_ref)

    dinvt = dinvt_ref[...]
    na = dinvt * (qa_ref[0] + qa_ref[1] + g2t_ref[0:1, :])
    nb = dinvt * (qb_ref[0] + qb_ref[1] + g2t_ref[1:2, :])
    ids = lax.broadcasted_iota(jnp.int32, (G, BLK), 0)
    m = (batch_ref[...] == ids).astype(jnp.float32)
    ra = jnp.sum(m * na, axis=1, keepdims=True)
    rb = jnp.sum(m * nb, axis=1, keepdims=True)
    out_ref[...] += jnp.concatenate([ra, rb], axis=1)
    cnt_ref[...] += jnp.sum(m, axis=1, keepdims=True)

    @pl.when(i == NPAD // BLK - 1)
    def _fin():
        cnt = cnt_ref[...]
        mean = out_ref[...] / jnp.maximum(cnt, 1.0)
        out_ref[...] = mean + b2_ref[...] * (cnt > 0.0)


def _e_call(qa, qb, g2t, dinv_t, batch_p, b2):
    return pl.pallas_call(
        _e_body,
        grid=(NPAD // BLK,),
        in_specs=[
            pl.BlockSpec((NC, 1, BLK), lambda i: (0, 0, i)),
            pl.BlockSpec((NC, 1, BLK), lambda i: (0, 0, i)),
            pl.BlockSpec((EMB, BLK), lambda i: (0, i)),
            pl.BlockSpec((1, BLK), lambda i: (0, i)),
            pl.BlockSpec((1, BLK), lambda i: (0, i)),
            pl.BlockSpec((1, EMB), lambda i: (0, 0)),
        ],
        out_specs=pl.BlockSpec((G, EMB), lambda i: (0, 0)),
        out_shape=jax.ShapeDtypeStruct((G, EMB), jnp.float32),
        scratch_shapes=[pltpu.VMEM((G, 1), jnp.float32)],
    )(qa, qb, g2t, dinv_t, batch_p, b2)


@jax.jit
def kernel(x, edge_index, batch, W1, b1, W2, b2):
    src = edge_index[0].astype(jnp.int32)
    dst = edge_index[1].astype(jnp.int32)
    pad = N + (jnp.arange(EPAD - E, dtype=jnp.int32) % (NPAD - N))
    src3 = jnp.concatenate([src, pad]).reshape(NW, CHUNKS, CH)
    dst3 = jnp.concatenate([dst, pad]).reshape(NW, CHUNKS, CH)
    x_p = jnp.pad(x.astype(jnp.float32), ((0, NPAD - N), (0, 0)))
    batch_p = jnp.pad(batch.astype(jnp.int32), (0, NPAD - N),
                      constant_values=G).reshape(1, NPAD)

    deg = _deg_call()(dst3)
    deg_t = jnp.transpose(deg.reshape(NC, NPAD))
    g1, dinv = _b_call(deg_t, x_p, W1)
    p = _make_agg(HIDDEN)(g1, src3, dst3)
    g2t = _d_call(p, g1, dinv, W2, b1.reshape(1, HIDDEN))
    qa, qb = _agg2e_call()(g2t[0], g2t[1], src3, dst3)
    dinv_t = jnp.transpose(dinv)
    out = _e_call(qa, qb, g2t, dinv_t, batch_p, b2.reshape(1, EMB))
    return out
